# Initial kernel scaffold; baseline (speedup 1.0000x reference)
import functools

import jax
import jax.numpy as jnp
from jax import lax
from jax.experimental import pallas as pl
from jax.experimental.pallas import tpu as pltpu
from jax.experimental.pallas import tpu_sc as plsc

N_USERS = 4000
N_ITEMS = 6000
N_NODES = N_USERS + N_ITEMS
N_EDGES = 320000
DIM = 128
HALF = 64
N_LAYERS = 3
BATCH_USERS = 1024

NC = 2
NS = 16
K = 128

N_PAD = 10240
ROWS_PT = N_PAD // NS
CHUNKS_PT = -(-N_EDGES // (NS * K))
E_PAD = NS * CHUNKS_PT * K
ITEMS_PAD = 6016
IROWS_PT = ITEMS_PAD // NS
U_PT = BATCH_USERS // NS


def _propagate_body(src2, dst2, w2, emb0, upair, light, uout,
                    accum, ssum, srcv, dstv, wv, rows, stage, sem):
    cid = lax.axis_index("c")
    sid = lax.axis_index("s")
    rbase = sid * ROWS_PT
    ebase = sid * CHUNKS_PT

    def zero_stage():
        @pl.loop(0, K)
        def _z(r):
            for c in range(HALF // 16):
                stage[r, pl.ds(c * 16, 16)] = jnp.zeros((16,), jnp.float32)

    zero_stage()

    for b in range(ROWS_PT // K):
        pltpu.sync_copy(emb0.at[pl.ds(cid * N_PAD + rbase + b * K, K)],
                        rows)
        pltpu.sync_copy(rows, ssum.at[pl.ds(rbase + b * K, K)])

    gather_src = emb0
    for layer in range(N_LAYERS):
        for b in range(ROWS_PT // K):
            pltpu.sync_copy(stage, accum.at[pl.ds(rbase + b * K, K)])
        plsc.subcore_barrier()

        @pl.loop(0, CHUNKS_PT)
        def _edge_chunk(g):
            row = ebase + g
            pltpu.sync_copy(src2.at[cid, row], srcv)
            pltpu.sync_copy(dst2.at[row], dstv)
            pltpu.sync_copy(w2.at[row], wv)
            pltpu.async_copy(gather_src.at[srcv], rows, sem).wait()

            @pl.loop(0, K)
            def _scale(r):
                w = wv[r]
                for c in range(HALF // 16):
                    sl = pl.ds(c * 16, 16)
                    rows[r, sl] = rows[r, sl] * w

            pltpu.sync_copy(rows, accum.at[dstv], add=True)

        plsc.subcore_barrier()

        last = layer == N_LAYERS - 1
        for b in range(ROWS_PT // K):
            sl = pl.ds(rbase + b * K, K)
            pltpu.sync_copy(accum.at[sl], rows)
            pltpu.sync_copy(ssum.at[sl], stage)

            @pl.loop(0, K)
            def _acc(r):
                for c in range(HALF // 16):
                    cs = pl.ds(c * 16, 16)
                    stage[r, cs] = stage[r, cs] + rows[r, cs]

            pltpu.sync_copy(stage, ssum.at[sl])
            if not last:
                pltpu.sync_copy(
                    rows, light.at[pl.ds(cid * N_PAD + rbase + b * K, K)])
        if not last:
            plsc.subcore_barrier()
            zero_stage()
        gather_src = light

    for b in range(ROWS_PT // K):
        sl = pl.ds(rbase + b * K, K)
        pltpu.sync_copy(ssum.at[sl], rows)

        @pl.loop(0, K)
        def _scale_out(r):
            for c in range(HALF // 16):
                cs = pl.ds(c * 16, 16)
                rows[r, cs] = rows[r, cs] * 0.25

        pltpu.sync_copy(rows, light.at[pl.ds(cid * N_PAD + rbase + b * K, K)])
    plsc.subcore_barrier()

    pltpu.sync_copy(upair.at[cid, pl.ds(sid * U_PT, U_PT)],
                    srcv.at[pl.ds(0, U_PT)])
    pltpu.async_copy(light.at[srcv.at[pl.ds(0, U_PT)]],
                     rows.at[pl.ds(0, U_PT)], sem).wait()
    pltpu.sync_copy(rows.at[pl.ds(0, U_PT)],
                    uout.at[pl.ds(cid * BATCH_USERS + sid * U_PT, U_PT)])


@jax.jit
def _propagate(src2, dst2, w2, emb0, upair):
    mesh = plsc.VectorSubcoreMesh(core_axis_name="c", subcore_axis_name="s")
    return pl.kernel(
        _propagate_body,
        out_type=(
            jax.ShapeDtypeStruct((NC * N_PAD, HALF), jnp.float32),
            jax.ShapeDtypeStruct((NC * BATCH_USERS, HALF), jnp.float32),
        ),
        mesh=mesh,
        scratch_types=[
            pltpu.VMEM_SHARED((N_PAD, HALF), jnp.float32),
            pltpu.VMEM_SHARED((N_PAD, HALF), jnp.float32),
            pltpu.VMEM((K,), jnp.int32),
            pltpu.VMEM((K,), jnp.int32),
            pltpu.VMEM((K,), jnp.float32),
            pltpu.VMEM((K, HALF), jnp.float32),
            pltpu.VMEM((K, HALF), jnp.float32),
            pltpu.SemaphoreType.DMA,
        ],
    )(src2, dst2, w2, emb0, upair)


def _rating_body(u0, u1, i0, i1, out):
    acc = jax.lax.dot_general(u0[...], i0[...], (((1,), (1,)), ((), ())),
                              preferred_element_type=jnp.float32)
    acc += jax.lax.dot_general(u1[...], i1[...], (((1,), (1,)), ((), ())),
                               preferred_element_type=jnp.float32)
    out[...] = 1.0 / (1.0 + jnp.exp(-acc))


@jax.jit
def _rating(u0, u1, i0, i1):
    m_blk = 128
    grid = (BATCH_USERS // m_blk,)
    return pl.pallas_call(
        _rating_body,
        grid=grid,
        in_specs=[
            pl.BlockSpec((m_blk, HALF), lambda i: (i, 0)),
            pl.BlockSpec((m_blk, HALF), lambda i: (i, 0)),
            pl.BlockSpec((ITEMS_PAD, HALF), lambda i: (0, 0)),
            pl.BlockSpec((ITEMS_PAD, HALF), lambda i: (0, 0)),
        ],
        out_specs=pl.BlockSpec((m_blk, ITEMS_PAD), lambda i: (i, 0)),
        out_shape=jax.ShapeDtypeStruct((BATCH_USERS, ITEMS_PAD), jnp.float32),
    )(u0, u1, i0, i1)


def kernel(user_emb, item_emb, edge_weight, edge_index, users):
    all_emb = jnp.concatenate([user_emb, item_emb], axis=0)
    all_emb = jnp.pad(all_emb, ((0, N_PAD - N_NODES), (0, 0)))
    emb0 = all_emb.reshape(N_PAD, NC, HALF).transpose(1, 0, 2)
    emb0 = emb0.reshape(NC * N_PAD, HALF)

    src = edge_index[0]
    dst = edge_index[1]
    pad_e = E_PAD - N_EDGES
    pad_rows = N_NODES + (jnp.arange(pad_e, dtype=jnp.int32)
                          % (N_PAD - N_NODES))
    src_p = jnp.concatenate([src, pad_rows])
    dst_p = jnp.concatenate([dst, pad_rows])
    w_p = jnp.concatenate([edge_weight, jnp.zeros((pad_e,), jnp.float32)])
    src2 = jnp.stack([src_p, src_p + N_PAD]).reshape(NC, -1, K)
    dst2 = dst_p.reshape(-1, K)
    w2 = w_p.reshape(-1, K)
    upair = jnp.stack([users, users + N_PAD])

    light, uout = _propagate(src2, dst2, w2, emb0, upair)

    i0 = light[N_USERS:N_USERS + ITEMS_PAD]
    i1 = light[N_PAD + N_USERS:N_PAD + N_USERS + ITEMS_PAD]
    u0 = uout[:BATCH_USERS]
    u1 = uout[BATCH_USERS:]
    rating = _rating(u0, u1, i0, i1)
    return rating[:, :N_ITEMS]


# speedup vs baseline: 1.9700x; 1.9700x over previous
import functools

import jax
import jax.numpy as jnp
from jax import lax
from jax.experimental import pallas as pl
from jax.experimental.pallas import tpu as pltpu
from jax.experimental.pallas import tpu_sc as plsc

N_USERS = 4000
N_ITEMS = 6000
N_NODES = N_USERS + N_ITEMS
N_EDGES = 320000
DIM = 128
HALF = 64
N_LAYERS = 3
BATCH_USERS = 1024

NC = 2
NS = 16
K = 128

N_PAD = 10240
ROWS_PT = N_PAD // NS
CHUNKS_PT = -(-N_EDGES // (NS * K))
E_PAD = NS * CHUNKS_PT * K
ITEMS_PAD = 6016
IROWS_PT = ITEMS_PAD // NS
U_PT = BATCH_USERS // NS


def _propagate_body(src2, dst2, w2, emb0, upair, light, uout,
                    accum, ssum, srcv, dstv, wv, rows, stage, sem):
    cid = lax.axis_index("c")
    sid = lax.axis_index("s")
    rbase = sid * ROWS_PT
    ebase = sid * CHUNKS_PT

    def zero_stage():
        @pl.loop(0, K)
        def _z(r):
            for c in range(HALF // 16):
                stage[r, pl.ds(c * 16, 16)] = jnp.zeros((16,), jnp.float32)

    zero_stage()

    for b in range(ROWS_PT // K):
        pltpu.sync_copy(emb0.at[pl.ds(cid * N_PAD + rbase + b * K, K)],
                        rows)
        pltpu.sync_copy(rows, ssum.at[pl.ds(rbase + b * K, K)])

    gather_src = emb0
    for layer in range(N_LAYERS):
        for b in range(ROWS_PT // K):
            pltpu.sync_copy(stage, accum.at[pl.ds(rbase + b * K, K)])
        plsc.subcore_barrier()

        @pl.loop(0, CHUNKS_PT)
        def _edge_chunk(g):
            row = ebase + g
            pltpu.sync_copy(src2.at[cid, row], srcv)
            pltpu.sync_copy(dst2.at[row], dstv)
            pltpu.sync_copy(w2.at[row], wv)
            pltpu.async_copy(gather_src.at[srcv], rows, sem).wait()

            @pl.loop(0, K // 16)
            def _scale(g):
                w16 = wv[pl.ds(g * 16, 16)]
                for j in range(16):
                    r = g * 16 + j
                    wj = w16[j]
                    for c in range(HALF // 16):
                        sl = pl.ds(c * 16, 16)
                        rows[r, sl] = rows[r, sl] * wj

            pltpu.sync_copy(rows, accum.at[dstv], add=True)

        plsc.subcore_barrier()

        last = layer == N_LAYERS - 1
        for b in range(ROWS_PT // K):
            sl = pl.ds(rbase + b * K, K)
            pltpu.sync_copy(accum.at[sl], rows)
            pltpu.sync_copy(ssum.at[sl], stage)

            @pl.loop(0, K)
            def _acc(r):
                for c in range(HALF // 16):
                    cs = pl.ds(c * 16, 16)
                    stage[r, cs] = stage[r, cs] + rows[r, cs]

            pltpu.sync_copy(stage, ssum.at[sl])
            if not last:
                pltpu.sync_copy(
                    rows, light.at[pl.ds(cid * N_PAD + rbase + b * K, K)])
        if not last:
            plsc.subcore_barrier()
            zero_stage()
        gather_src = light

    for b in range(ROWS_PT // K):
        sl = pl.ds(rbase + b * K, K)
        pltpu.sync_copy(ssum.at[sl], rows)

        @pl.loop(0, K)
        def _scale_out(r):
            for c in range(HALF // 16):
                cs = pl.ds(c * 16, 16)
                rows[r, cs] = rows[r, cs] * 0.25

        pltpu.sync_copy(rows, light.at[pl.ds(cid * N_PAD + rbase + b * K, K)])
    plsc.subcore_barrier()

    pltpu.sync_copy(upair.at[cid, pl.ds(sid * U_PT, U_PT)],
                    srcv.at[pl.ds(0, U_PT)])
    pltpu.async_copy(light.at[srcv.at[pl.ds(0, U_PT)]],
                     rows.at[pl.ds(0, U_PT)], sem).wait()
    pltpu.sync_copy(rows.at[pl.ds(0, U_PT)],
                    uout.at[pl.ds(cid * BATCH_USERS + sid * U_PT, U_PT)])


@jax.jit
def _propagate(src2, dst2, w2, emb0, upair):
    mesh = plsc.VectorSubcoreMesh(core_axis_name="c", subcore_axis_name="s")
    return pl.kernel(
        _propagate_body,
        out_type=(
            jax.ShapeDtypeStruct((NC * N_PAD, HALF), jnp.float32),
            jax.ShapeDtypeStruct((NC * BATCH_USERS, HALF), jnp.float32),
        ),
        mesh=mesh,
        scratch_types=[
            pltpu.VMEM_SHARED((N_PAD, HALF), jnp.float32),
            pltpu.VMEM_SHARED((N_PAD, HALF), jnp.float32),
            pltpu.VMEM((K,), jnp.int32),
            pltpu.VMEM((K,), jnp.int32),
            pltpu.VMEM((K,), jnp.float32),
            pltpu.VMEM((K, HALF), jnp.float32),
            pltpu.VMEM((K, HALF), jnp.float32),
            pltpu.SemaphoreType.DMA,
        ],
        compiler_params=pltpu.CompilerParams(use_tc_tiling_on_sc=False),
    )(src2, dst2, w2, emb0, upair)


def _rating_body(u0, u1, i0, i1, out):
    acc = jax.lax.dot_general(u0[...], i0[...], (((1,), (1,)), ((), ())),
                              preferred_element_type=jnp.float32)
    acc += jax.lax.dot_general(u1[...], i1[...], (((1,), (1,)), ((), ())),
                               preferred_element_type=jnp.float32)
    out[...] = 1.0 / (1.0 + jnp.exp(-acc))


@jax.jit
def _rating(u0, u1, i0, i1):
    m_blk = 128
    grid = (BATCH_USERS // m_blk,)
    return pl.pallas_call(
        _rating_body,
        grid=grid,
        in_specs=[
            pl.BlockSpec((m_blk, HALF), lambda i: (i, 0)),
            pl.BlockSpec((m_blk, HALF), lambda i: (i, 0)),
            pl.BlockSpec((ITEMS_PAD, HALF), lambda i: (0, 0)),
            pl.BlockSpec((ITEMS_PAD, HALF), lambda i: (0, 0)),
        ],
        out_specs=pl.BlockSpec((m_blk, ITEMS_PAD), lambda i: (i, 0)),
        out_shape=jax.ShapeDtypeStruct((BATCH_USERS, ITEMS_PAD), jnp.float32),
    )(u0, u1, i0, i1)


def kernel(user_emb, item_emb, edge_weight, edge_index, users):
    all_emb = jnp.concatenate([user_emb, item_emb], axis=0)
    all_emb = jnp.pad(all_emb, ((0, N_PAD - N_NODES), (0, 0)))
    emb0 = all_emb.reshape(N_PAD, NC, HALF).transpose(1, 0, 2)
    emb0 = emb0.reshape(NC * N_PAD, HALF)

    src = edge_index[0]
    dst = edge_index[1]
    pad_e = E_PAD - N_EDGES
    pad_rows = N_NODES + (jnp.arange(pad_e, dtype=jnp.int32)
                          % (N_PAD - N_NODES))
    src_p = jnp.concatenate([src, pad_rows])
    dst_p = jnp.concatenate([dst, pad_rows])
    w_p = jnp.concatenate([edge_weight, jnp.zeros((pad_e,), jnp.float32)])
    src2 = jnp.stack([src_p, src_p + N_PAD]).reshape(NC, -1, K)
    dst2 = dst_p.reshape(-1, K)
    w2 = w_p.reshape(-1, K)
    upair = jnp.stack([users, users + N_PAD])

    light, uout = _propagate(src2, dst2, w2, emb0, upair)

    i0 = light[N_USERS:N_USERS + ITEMS_PAD]
    i1 = light[N_PAD + N_USERS:N_PAD + N_USERS + ITEMS_PAD]
    u0 = uout[:BATCH_USERS]
    u1 = uout[BATCH_USERS:]
    rating = _rating(u0, u1, i0, i1)
    return rating[:, :N_ITEMS]
